# Initial kernel scaffold; baseline (speedup 1.0000x reference)
import jax
import jax.numpy as jnp
from jax.experimental import pallas as pl


def kernel(message, x, edge_index, edge_attr, Wl1, bl1, Wr1, br1, We1, att1, bo1, Wl2, bl2, Wr2, br2, We2, att2, bo2, Wfc, bfc):
    raise NotImplementedError("write your pallas kernel here")


# speedup vs baseline: 76.8987x; 76.8987x over previous
import functools

import jax
import jax.numpy as jnp
from jax import lax
from jax.experimental import pallas as pl
from jax.experimental.pallas import tpu as pltpu
from jax.experimental.pallas import tpu_sc as plsc

N = 50000
E = 800000

NC = 2
NS = 16
NW = NC * NS

CHA = 1600
NCH_A = E // CHA
CHC = 3200
NCH_C = E // CHC
CHC_PER_SC = NCH_C // NC

BR_B = 4000
RB_D = 400

_mesh = plsc.VectorSubcoreMesh(core_axis_name="c", subcore_axis_name="s")


@functools.partial(
    pl.kernel,
    mesh=_mesh,
    compiler_params=pltpu.CompilerParams(needs_layout_passes=False, use_tc_tiling_on_sc=False),
    out_type=jax.ShapeDtypeStruct((E, 8), jnp.float32),
    scratch_types=[
        pltpu.VMEM((2 * N,), jnp.float32),
        pltpu.VMEM((CHA,), jnp.int32),
        pltpu.VMEM((CHA,), jnp.int32),
        pltpu.VMEM((CHA,), jnp.float32),
        pltpu.VMEM((CHA, 8), jnp.float32),
    ],
)
def _gather_stage(x_hbm, src_hbm, dst_hbm, ea_hbm, g_hbm,
                  x_v, src_v, dst_v, ea_v, rows_v):
    c = lax.axis_index("c")
    s = lax.axis_index("s")
    wid = s * NC + c
    pltpu.sync_copy(x_hbm, x_v)

    col0 = jnp.zeros((16,), jnp.int32)
    col1 = jnp.ones((16,), jnp.int32)
    ones_f = jnp.ones((16,), jnp.float32)
    iota = lax.iota(jnp.int32, 16)

    def do_chunk(cid):
        base = cid * CHA
        pltpu.sync_copy(src_hbm.at[pl.ds(base, CHA)], src_v)
        pltpu.sync_copy(dst_hbm.at[pl.ds(base, CHA)], dst_v)
        pltpu.sync_copy(ea_hbm.at[pl.ds(base, CHA)], ea_v)

        def grp(j, carry):
            off = j * 16
            si = src_v[pl.ds(off, 16)]
            di = dst_v[pl.ds(off, 16)]
            eav = ea_v[pl.ds(off, 16)]
            si2 = si + si
            di2 = di + di
            xs0 = plsc.load_gather(x_v, [si2])
            xs1 = plsc.load_gather(x_v, [si2 + 1])
            xd0 = plsc.load_gather(x_v, [di2])
            xd1 = plsc.load_gather(x_v, [di2 + 1])
            rowi = off + iota
            plsc.store_scatter(rows_v, [rowi, col0], xs0)
            plsc.store_scatter(rows_v, [rowi, col1], xs1)
            plsc.store_scatter(rows_v, [rowi, col0 + 2], xd0)
            plsc.store_scatter(rows_v, [rowi, col1 + 2], xd1)
            plsc.store_scatter(rows_v, [rowi, col0 + 4], eav)
            plsc.store_scatter(rows_v, [rowi, col1 + 4], ones_f)
            return carry

        lax.fori_loop(0, CHA // 16, grp, 0)
        pltpu.sync_copy(rows_v, g_hbm.at[pl.ds(base, CHA)])

    def chunk_iter(i, carry):
        cid = wid + i * NW

        @pl.when(cid < NCH_A)
        def _go():
            do_chunk(cid)

        return carry

    lax.fori_loop(0, (NCH_A + NW - 1) // NW, chunk_iter, 0)


@functools.partial(
    pl.kernel,
    mesh=_mesh,
    compiler_params=pltpu.CompilerParams(needs_layout_passes=False, use_tc_tiling_on_sc=False),
    out_type=(
        jax.ShapeDtypeStruct((N, 8), jnp.float32),
        jax.ShapeDtypeStruct((N, 8), jnp.float32),
    ),
    scratch_types=[
        pltpu.VMEM_SHARED((N, 8), jnp.float32),
        pltpu.VMEM((CHC, 8), jnp.float32),
        pltpu.VMEM((CHC,), jnp.int32),
    ],
)
def _scatter_stage(p_hbm, dst_hbm, zeros_hbm, t0_hbm, t1_hbm,
                   table_sh, rows_v, idx_v):
    c = lax.axis_index("c")
    s = lax.axis_index("s")
    zr = N // NS
    pltpu.sync_copy(zeros_hbm.at[pl.ds(s * zr, zr)],
                    table_sh.at[pl.ds(s * zr, zr)])
    plsc.subcore_barrier()

    def chunk_iter(i, carry):
        cid = c * CHC_PER_SC + s + NS * i

        @pl.when(cid < (c + 1) * CHC_PER_SC)
        def _go():
            base = cid * CHC
            pltpu.sync_copy(p_hbm.at[pl.ds(base, CHC)], rows_v)
            pltpu.sync_copy(dst_hbm.at[pl.ds(base, CHC)], idx_v)
            pltpu.sync_copy(rows_v, table_sh.at[idx_v], add=True)

        return carry

    lax.fori_loop(0, (CHC_PER_SC + NS - 1) // NS, chunk_iter, 0)
    plsc.subcore_barrier()

    @pl.when(c == 0)
    def _():
        pltpu.sync_copy(table_sh.at[pl.ds(s * zr, zr)],
                        t0_hbm.at[pl.ds(s * zr, zr)])

    @pl.when(c == 1)
    def _():
        pltpu.sync_copy(table_sh.at[pl.ds(s * zr, zr)],
                        t1_hbm.at[pl.ds(s * zr, zr)])


def _edge_dense_body(g_ref, coef_ref, a2_ref, b8_ref, p_ref):
    g = g_ref[...]
    z = jnp.dot(g, coef_ref[...], preferred_element_type=jnp.float32)
    l = jnp.maximum(z, 0.2 * z)
    alpha8 = jnp.dot(l, a2_ref[...], preferred_element_type=jnp.float32)
    q = jnp.dot(g, b8_ref[...], preferred_element_type=jnp.float32)
    p_ref[...] = jnp.exp(alpha8) * q


def _node_dense_body(t0_ref, t1_ref, m_ref, s_ref, bo_ref, msg_ref,
                     wfc_ref, bfc_ref, exps_ref, colsum_ref):
    i = pl.program_id(0)
    t = t0_ref[...] + t1_ref[...]
    numer = jnp.dot(t, m_ref[...], preferred_element_type=jnp.float32)
    denom = jnp.dot(t, s_ref[...], preferred_element_type=jnp.float32) + 1e-16
    h = jnp.maximum(numer / denom + bo_ref[...], 0.0)
    e1 = jnp.dot(msg_ref[...], wfc_ref[...],
                 preferred_element_type=jnp.float32) + bfc_ref[...]
    d1 = lax.dot_general(h, e1, (((1,), (1,)), ((), ())),
                         preferred_element_type=jnp.float32)
    expd = jnp.exp(d1)
    exps_ref[...] = expd

    @pl.when(i == 0)
    def _():
        colsum_ref[...] = jnp.zeros_like(colsum_ref)

    colsum_ref[...] += jnp.sum(expd, axis=0, keepdims=True)


def _normalize_body(exps_ref, colsum_ref, out_ref):
    p = exps_ref[...] / colsum_ref[...]
    out_ref[...] = jnp.concatenate([p, p], axis=1)


def kernel(message, x, edge_index, edge_attr, Wl1, bl1, Wr1, br1, We1, att1,
           bo1, Wl2, bl2, Wr2, br2, We2, att2, bo2, Wfc, bfc):
    f32 = jnp.float32
    src = edge_index[0].astype(jnp.int32)
    dst = edge_index[1].astype(jnp.int32)
    ea = edge_attr.reshape(E)

    coef8 = jnp.concatenate(
        [Wl2, Wr2, We2, (bl2 + br2)[None], jnp.zeros((2, 64), f32)], axis=0)
    attf = att2.reshape(64)
    hmask = jnp.arange(64) < 32
    a0 = jnp.where(hmask, attf, 0.0)
    a1 = jnp.where(hmask, 0.0, attf)
    a2_8 = jnp.stack([a0, a1, a0, a1, a0, a1, a0, a1], axis=1)
    b8 = jnp.zeros((8, 8), f32)
    b8 = b8.at[5, 0].set(1.0).at[5, 1].set(1.0)
    b8 = b8.at[0, 2].set(1.0).at[0, 3].set(1.0)
    b8 = b8.at[1, 4].set(1.0).at[1, 5].set(1.0)
    b8 = b8.at[5, 6].set(1.0).at[5, 7].set(1.0)

    m8 = jnp.zeros((8, 64), f32)
    m8 = m8.at[0].set(jnp.where(hmask, bl2, 0.0))
    m8 = m8.at[1].set(jnp.where(hmask, 0.0, bl2))
    m8 = m8.at[2].set(jnp.where(hmask, Wl2[0], 0.0))
    m8 = m8.at[3].set(jnp.where(hmask, 0.0, Wl2[0]))
    m8 = m8.at[4].set(jnp.where(hmask, Wl2[1], 0.0))
    m8 = m8.at[5].set(jnp.where(hmask, 0.0, Wl2[1]))
    s8 = jnp.zeros((8, 64), f32)
    s8 = s8.at[0].set(jnp.where(hmask, 1.0, 0.0))
    s8 = s8.at[1].set(jnp.where(hmask, 0.0, 1.0))

    zeros_tab = jnp.zeros((N, 8), f32)

    g = _gather_stage(x.reshape(2 * N), src, dst, ea)

    p = pl.pallas_call(
        _edge_dense_body,
        grid=(E // BR_B,),
        in_specs=[
            pl.BlockSpec((BR_B, 8), lambda i: (i, 0)),
            pl.BlockSpec((8, 64), lambda i: (0, 0)),
            pl.BlockSpec((64, 8), lambda i: (0, 0)),
            pl.BlockSpec((8, 8), lambda i: (0, 0)),
        ],
        out_specs=pl.BlockSpec((BR_B, 8), lambda i: (i, 0)),
        out_shape=jax.ShapeDtypeStruct((E, 8), f32),
    )(g, coef8, a2_8, b8)

    t0, t1 = _scatter_stage(p, dst, zeros_tab)

    exps, colsum = pl.pallas_call(
        _node_dense_body,
        grid=(N // RB_D,),
        in_specs=[
            pl.BlockSpec((RB_D, 8), lambda i: (i, 0)),
            pl.BlockSpec((RB_D, 8), lambda i: (i, 0)),
            pl.BlockSpec((8, 64), lambda i: (0, 0)),
            pl.BlockSpec((8, 64), lambda i: (0, 0)),
            pl.BlockSpec((1, 64), lambda i: (0, 0)),
            pl.BlockSpec((16, 128), lambda i: (0, 0)),
            pl.BlockSpec((128, 64), lambda i: (0, 0)),
            pl.BlockSpec((1, 64), lambda i: (0, 0)),
        ],
        out_specs=[
            pl.BlockSpec((RB_D, 16), lambda i: (i, 0)),
            pl.BlockSpec((1, 16), lambda i: (0, 0)),
        ],
        out_shape=[
            jax.ShapeDtypeStruct((N, 16), f32),
            jax.ShapeDtypeStruct((1, 16), f32),
        ],
    )(t0, t1, m8, s8, bo2[None], message, Wfc, bfc[None])

    probs = pl.pallas_call(
        _normalize_body,
        grid=(N // RB_D,),
        in_specs=[
            pl.BlockSpec((RB_D, 16), lambda i: (i, 0)),
            pl.BlockSpec((1, 16), lambda i: (0, 0)),
        ],
        out_specs=pl.BlockSpec((RB_D, 32), lambda i: (i, 0)),
        out_shape=jax.ShapeDtypeStruct((N, 32), f32),
    )(exps, colsum)

    return probs


# speedup vs baseline: 178.9858x; 2.3276x over previous
import functools

import jax
import jax.numpy as jnp
from jax import lax
from jax.experimental import pallas as pl
from jax.experimental.pallas import tpu as pltpu
from jax.experimental.pallas import tpu_sc as plsc

N = 50000
E = 800000

NC = 2
NS = 16
NW = NC * NS

CHA = 1600
NCH_A = E // CHA
CHC = 3200
NCH_C = E // CHC
CHC_PER_SC = NCH_C // NC

BR_B = 400
RB_D = 400

_mesh = plsc.VectorSubcoreMesh(core_axis_name="c", subcore_axis_name="s")


@functools.partial(
    pl.kernel,
    mesh=_mesh,
    compiler_params=pltpu.CompilerParams(needs_layout_passes=False, use_tc_tiling_on_sc=False),
    out_type=jax.ShapeDtypeStruct((E, 8), jnp.float32),
    scratch_types=[
        pltpu.VMEM((2 * N,), jnp.float32),
        pltpu.VMEM((CHA,), jnp.int32),
        pltpu.VMEM((CHA,), jnp.int32),
        pltpu.VMEM((CHA,), jnp.float32),
        pltpu.VMEM((CHA, 8), jnp.float32),
    ],
)
def _gather_stage(x_hbm, src_hbm, dst_hbm, ea_hbm, g_hbm,
                  x_v, src_v, dst_v, ea_v, rows_v):
    c = lax.axis_index("c")
    s = lax.axis_index("s")
    wid = s * NC + c
    pltpu.sync_copy(x_hbm, x_v)

    col0 = jnp.zeros((16,), jnp.int32)
    col1 = jnp.ones((16,), jnp.int32)
    ones_f = jnp.ones((16,), jnp.float32)
    iota = lax.iota(jnp.int32, 16)

    def do_chunk(cid):
        base = cid * CHA
        pltpu.sync_copy(src_hbm.at[pl.ds(base, CHA)], src_v)
        pltpu.sync_copy(dst_hbm.at[pl.ds(base, CHA)], dst_v)
        pltpu.sync_copy(ea_hbm.at[pl.ds(base, CHA)], ea_v)

        def grp(j, carry):
            off = j * 16
            si = src_v[pl.ds(off, 16)]
            di = dst_v[pl.ds(off, 16)]
            eav = ea_v[pl.ds(off, 16)]
            si2 = si + si
            di2 = di + di
            xs0 = plsc.load_gather(x_v, [si2])
            xs1 = plsc.load_gather(x_v, [si2 + 1])
            xd0 = plsc.load_gather(x_v, [di2])
            xd1 = plsc.load_gather(x_v, [di2 + 1])
            rowi = off + iota
            plsc.store_scatter(rows_v, [rowi, col0], xs0)
            plsc.store_scatter(rows_v, [rowi, col1], xs1)
            plsc.store_scatter(rows_v, [rowi, col0 + 2], xd0)
            plsc.store_scatter(rows_v, [rowi, col1 + 2], xd1)
            plsc.store_scatter(rows_v, [rowi, col0 + 4], eav)
            plsc.store_scatter(rows_v, [rowi, col1 + 4], ones_f)
            return carry

        lax.fori_loop(0, CHA // 16, grp, 0)
        pltpu.sync_copy(rows_v, g_hbm.at[pl.ds(base, CHA)])

    def chunk_iter(i, carry):
        cid = wid + i * NW

        @pl.when(cid < NCH_A)
        def _go():
            do_chunk(cid)

        return carry

    lax.fori_loop(0, (NCH_A + NW - 1) // NW, chunk_iter, 0)


@functools.partial(
    pl.kernel,
    mesh=_mesh,
    compiler_params=pltpu.CompilerParams(needs_layout_passes=False, use_tc_tiling_on_sc=False),
    out_type=(
        jax.ShapeDtypeStruct((N, 8), jnp.float32),
        jax.ShapeDtypeStruct((N, 8), jnp.float32),
    ),
    scratch_types=[
        pltpu.VMEM_SHARED((N, 8), jnp.float32),
        pltpu.VMEM((CHC, 8), jnp.float32),
        pltpu.VMEM((CHC,), jnp.int32),
    ],
)
def _scatter_stage(p_hbm, dst_hbm, zeros_hbm, t0_hbm, t1_hbm,
                   table_sh, rows_v, idx_v):
    c = lax.axis_index("c")
    s = lax.axis_index("s")
    zr = N // NS
    pltpu.sync_copy(zeros_hbm.at[pl.ds(s * zr, zr)],
                    table_sh.at[pl.ds(s * zr, zr)])
    plsc.subcore_barrier()

    def chunk_iter(i, carry):
        cid = c * CHC_PER_SC + s + NS * i

        @pl.when(cid < (c + 1) * CHC_PER_SC)
        def _go():
            base = cid * CHC
            pltpu.sync_copy(p_hbm.at[pl.ds(base, CHC)], rows_v)
            pltpu.sync_copy(dst_hbm.at[pl.ds(base, CHC)], idx_v)
            pltpu.sync_copy(rows_v, table_sh.at[idx_v], add=True)

        return carry

    lax.fori_loop(0, (CHC_PER_SC + NS - 1) // NS, chunk_iter, 0)
    plsc.subcore_barrier()

    @pl.when(c == 0)
    def _():
        pltpu.sync_copy(table_sh.at[pl.ds(s * zr, zr)],
                        t0_hbm.at[pl.ds(s * zr, zr)])

    @pl.when(c == 1)
    def _():
        pltpu.sync_copy(table_sh.at[pl.ds(s * zr, zr)],
                        t1_hbm.at[pl.ds(s * zr, zr)])


def _edge_dense_body(g_ref, kb_ref, ab_ref, bb_ref, p_ref):
    g = g_ref[...]
    z = jnp.dot(g, kb_ref[...], preferred_element_type=jnp.float32)
    l = jnp.maximum(z, 0.2 * z)
    ap = jnp.dot(l, ab_ref[...], preferred_element_type=jnp.float32)
    q = jnp.dot(g, bb_ref[...], preferred_element_type=jnp.float32)
    p_ref[...] = jnp.exp(ap) * q


def _node_dense_body(t0_ref, t1_ref, m_ref, s_ref, bo_ref, msg_ref,
                     wfc_ref, bfc_ref, exps_ref, colsum_ref):
    i = pl.program_id(0)
    t = t0_ref[...] + t1_ref[...]
    numer = jnp.dot(t, m_ref[...], preferred_element_type=jnp.float32)
    denom = jnp.dot(t, s_ref[...], preferred_element_type=jnp.float32) + 1e-16
    h = jnp.maximum(numer / denom + bo_ref[...], 0.0)
    e1 = jnp.dot(msg_ref[...], wfc_ref[...],
                 preferred_element_type=jnp.float32) + bfc_ref[...]
    d1 = lax.dot_general(h, e1, (((1,), (1,)), ((), ())),
                         preferred_element_type=jnp.float32)
    expd = jnp.exp(d1)
    exps_ref[...] = expd

    @pl.when(i == 0)
    def _():
        colsum_ref[...] = jnp.zeros_like(colsum_ref)

    colsum_ref[...] += jnp.sum(expd, axis=0, keepdims=True)


def _normalize_body(exps_ref, colsum_ref, out_ref):
    p = exps_ref[...] / colsum_ref[...]
    out_ref[...] = jnp.concatenate([p, p], axis=1)


def kernel(message, x, edge_index, edge_attr, Wl1, bl1, Wr1, br1, We1, att1,
           bo1, Wl2, bl2, Wr2, br2, We2, att2, bo2, Wfc, bfc):
    f32 = jnp.float32
    src = edge_index[0].astype(jnp.int32)
    dst = edge_index[1].astype(jnp.int32)
    ea = edge_attr.reshape(E)

    coef8 = jnp.concatenate(
        [Wl2, Wr2, We2, (bl2 + br2)[None], jnp.zeros((2, 64), f32)], axis=0)
    attf = att2.reshape(64)
    hmask = jnp.arange(64) < 32
    a0 = jnp.where(hmask, attf, 0.0)
    a1 = jnp.where(hmask, 0.0, attf)
    a2_8 = jnp.stack([a0, a1, a0, a1, a0, a1, a0, a1], axis=1)
    b8 = jnp.zeros((8, 8), f32)
    b8 = b8.at[5, 0].set(1.0).at[5, 1].set(1.0)
    b8 = b8.at[0, 2].set(1.0).at[0, 3].set(1.0)
    b8 = b8.at[1, 4].set(1.0).at[1, 5].set(1.0)
    b8 = b8.at[5, 6].set(1.0).at[5, 7].set(1.0)

    m8 = jnp.zeros((8, 64), f32)
    m8 = m8.at[0].set(jnp.where(hmask, bl2, 0.0))
    m8 = m8.at[1].set(jnp.where(hmask, 0.0, bl2))
    m8 = m8.at[2].set(jnp.where(hmask, Wl2[0], 0.0))
    m8 = m8.at[3].set(jnp.where(hmask, 0.0, Wl2[0]))
    m8 = m8.at[4].set(jnp.where(hmask, Wl2[1], 0.0))
    m8 = m8.at[5].set(jnp.where(hmask, 0.0, Wl2[1]))
    s8 = jnp.zeros((8, 64), f32)
    s8 = s8.at[0].set(jnp.where(hmask, 1.0, 0.0))
    s8 = s8.at[1].set(jnp.where(hmask, 0.0, 1.0))

    zeros_tab = jnp.zeros((N, 8), f32)

    g = _gather_stage(x.reshape(2 * N), src, dst, ea)

    g16 = g.reshape(E // 16, 128)
    eye16 = jnp.eye(16, dtype=f32)
    kb = jnp.kron(eye16, coef8)
    ab = jnp.kron(eye16, a2_8)
    bb = jnp.kron(eye16, b8)

    p16 = pl.pallas_call(
        _edge_dense_body,
        grid=(E // 16 // BR_B,),
        in_specs=[
            pl.BlockSpec((BR_B, 128), lambda i: (i, 0)),
            pl.BlockSpec((128, 1024), lambda i: (0, 0)),
            pl.BlockSpec((1024, 128), lambda i: (0, 0)),
            pl.BlockSpec((128, 128), lambda i: (0, 0)),
        ],
        out_specs=pl.BlockSpec((BR_B, 128), lambda i: (i, 0)),
        out_shape=jax.ShapeDtypeStruct((E // 16, 128), f32),
    )(g16, kb, ab, bb)

    t0, t1 = _scatter_stage(p16.reshape(E, 8), dst, zeros_tab)

    exps, colsum = pl.pallas_call(
        _node_dense_body,
        grid=(N // RB_D,),
        in_specs=[
            pl.BlockSpec((RB_D, 8), lambda i: (i, 0)),
            pl.BlockSpec((RB_D, 8), lambda i: (i, 0)),
            pl.BlockSpec((8, 64), lambda i: (0, 0)),
            pl.BlockSpec((8, 64), lambda i: (0, 0)),
            pl.BlockSpec((1, 64), lambda i: (0, 0)),
            pl.BlockSpec((16, 128), lambda i: (0, 0)),
            pl.BlockSpec((128, 64), lambda i: (0, 0)),
            pl.BlockSpec((1, 64), lambda i: (0, 0)),
        ],
        out_specs=[
            pl.BlockSpec((RB_D, 16), lambda i: (i, 0)),
            pl.BlockSpec((1, 16), lambda i: (0, 0)),
        ],
        out_shape=[
            jax.ShapeDtypeStruct((N, 16), f32),
            jax.ShapeDtypeStruct((1, 16), f32),
        ],
    )(t0, t1, m8, s8, bo2[None], message, Wfc, bfc[None])

    probs = pl.pallas_call(
        _normalize_body,
        grid=(N // RB_D,),
        in_specs=[
            pl.BlockSpec((RB_D, 16), lambda i: (i, 0)),
            pl.BlockSpec((1, 16), lambda i: (0, 0)),
        ],
        out_specs=pl.BlockSpec((RB_D, 32), lambda i: (i, 0)),
        out_shape=jax.ShapeDtypeStruct((N, 32), f32),
    )(exps, colsum)

    return probs
